# Initial kernel scaffold; baseline (speedup 1.0000x reference)
import jax
import jax.numpy as jnp
from jax.experimental import pallas as pl


def kernel(x, edge_index, edge_weight, W1, b1, W2, b2, W_lin, b_lin):
    raise NotImplementedError("write your pallas kernel here")


# speedup vs baseline: 3.4729x; 3.4729x over previous
import functools

import jax
import jax.numpy as jnp
from jax import lax
from jax.experimental import pallas as pl
from jax.experimental.pallas import tpu as pltpu
from jax.experimental.pallas import tpu_sc as plsc

MIN_NORM = 1e-15
EPS = 1e-7
MAX_NORM = 1e6

NC = 2
NS = 16
LANES = 16


def _col0_mask(v):
    col = lax.broadcasted_iota(jnp.int32, v.shape, 1)
    return col == 0


def _mask0(v):
    return jnp.where(_col0_mask(v), 0.0, v)


def _sinh(t):
    return 0.5 * (jnp.exp(t) - jnp.exp(-t))


def _arccosh(z):
    inv = 1.0 / z
    return jnp.log(z) + jnp.log1p(jnp.sqrt(jnp.maximum(1.0 - inv * inv, 0.0)))


def _rownorm(y):
    return jnp.maximum(jnp.sqrt(jnp.sum(y * y, axis=1, keepdims=True)), MIN_NORM)


def _proj(x):
    y = _mask0(x)
    x0 = jnp.sqrt(jnp.maximum(1.0 + jnp.sum(y * y, axis=1, keepdims=True), MIN_NORM))
    return jnp.where(_col0_mask(x), x0, x)


def _expmap0_proj(u):
    y = _mask0(u)
    yn = _rownorm(y)
    rest = (_sinh(yn) / yn) * y
    return _proj(rest)


def _logmap0(x):
    y = _mask0(x)
    yn = _rownorm(y)
    theta = jnp.maximum(x[:, 0:1], 1.0 + EPS)
    return (_arccosh(theta) / yn) * y


def _mobius_add_bias(x, hyp_bias):
    u = _logmap0(hyp_bias)
    x0 = x[:, 0:1]
    y = _mask0(x)
    yn = _rownorm(y)
    yu = y / yn
    v = jnp.where(_col0_mask(x), -yn, (1.0 - x0) * yu)
    alpha = jnp.sum(yu * _mask0(u), axis=1, keepdims=True)
    res = u - alpha * v
    ux = jnp.sum(y * _mask0(res), axis=1, keepdims=True)
    pt = jnp.where(_col0_mask(x), ux / jnp.maximum(x0, EPS), res)
    mink = jnp.sum(pt * pt, axis=1, keepdims=True) - 2.0 * pt[:, 0:1] * pt[:, 0:1]
    normu = jnp.minimum(jnp.sqrt(jnp.maximum(mink, EPS)), MAX_NORM)
    theta = jnp.maximum(normu, MIN_NORM)
    ch = 0.5 * (jnp.exp(theta) + jnp.exp(-theta))
    r = ch * x + (_sinh(theta) / theta) * pt
    return _proj(r)


def _hyp_bias(b_row):
    return _expmap0_proj(_mask0(b_row))


def _stage_a_body(x_ref, w1t_ref, b1_ref, o_ref):
    xb = x_ref[...]
    y = _mask0(xb)
    yn = _rownorm(y)
    rest = (_sinh(yn) / yn) * y
    x0 = jnp.sqrt(jnp.maximum(1.0 + jnp.sum(rest * rest, axis=1, keepdims=True),
                              MIN_NORM))
    yn2 = _rownorm(rest)
    theta = jnp.maximum(x0, 1.0 + EPS)
    u = (_arccosh(theta) / yn2) * rest
    mu = jnp.dot(u, w1t_ref[...], preferred_element_type=jnp.float32)
    res = _expmap0_proj(mu)
    res = _mobius_add_bias(res, _hyp_bias(b1_ref[...]))
    o_ref[...] = _logmap0(res)


def _stage_a(x, w1t, b1row, d1, bn=400):
    n = x.shape[0]
    return pl.pallas_call(
        _stage_a_body,
        grid=(n // bn,),
        in_specs=[
            pl.BlockSpec((bn, x.shape[1]), lambda i: (i, 0)),
            pl.BlockSpec(w1t.shape, lambda i: (0, 0)),
            pl.BlockSpec(b1row.shape, lambda i: (0, 0)),
        ],
        out_specs=pl.BlockSpec((bn, d1), lambda i: (i, 0)),
        out_shape=jax.ShapeDtypeStruct((n, d1), jnp.float32),
    )(x, w1t, b1row)


def _stage_b_body(p0_ref, p1_ref, w2t_ref, b2_ref, o_ref):
    support = p0_ref[...] + p1_ref[...]
    h = _expmap0_proj(support)
    xt = jax.nn.relu(_logmap0(h))
    h2 = _expmap0_proj(xt)
    u = _logmap0(h2)
    mu = jnp.dot(u, w2t_ref[...], preferred_element_type=jnp.float32)
    res = _expmap0_proj(mu)
    res = _mobius_add_bias(res, _hyp_bias(b2_ref[...]))
    o_ref[...] = _logmap0(res)


def _stage_b(p0, p1, w2t, b2row, d2, bn=1000):
    n = p0.shape[0]
    return pl.pallas_call(
        _stage_b_body,
        grid=(n // bn,),
        in_specs=[
            pl.BlockSpec((bn, p0.shape[1]), lambda i: (i, 0)),
            pl.BlockSpec((bn, p0.shape[1]), lambda i: (i, 0)),
            pl.BlockSpec(w2t.shape, lambda i: (0, 0)),
            pl.BlockSpec(b2row.shape, lambda i: (0, 0)),
        ],
        out_specs=pl.BlockSpec((bn, d2), lambda i: (i, 0)),
        out_shape=jax.ShapeDtypeStruct((n, d2), jnp.float32),
    )(p0, p1, w2t, b2row)


def _stage_c_body(q0_ref, q1_ref, wlt_ref, bl_ref, o_ref, *, n_cls):
    support = q0_ref[...] + q1_ref[...]
    h = _expmap0_proj(support)
    xt = jax.nn.relu(_logmap0(h))
    h2 = _expmap0_proj(xt)
    ht = _logmap0(h2)
    logits = jnp.dot(ht, wlt_ref[...], preferred_element_type=jnp.float32)
    logits = jax.nn.relu(logits + bl_ref[...])
    col = lax.broadcasted_iota(jnp.int32, logits.shape, 1)
    masked = jnp.where(col < n_cls, logits, -jnp.inf)
    m = jnp.max(masked, axis=1, keepdims=True)
    z = masked - m
    lse = jnp.log(jnp.sum(jnp.where(col < n_cls, jnp.exp(z), 0.0),
                          axis=1, keepdims=True))
    o_ref[...] = (z - lse)[:, :n_cls]


def _stage_c(q0, q1, wlt, blrow, n_cls, bn=1000):
    n = q0.shape[0]
    return pl.pallas_call(
        functools.partial(_stage_c_body, n_cls=n_cls),
        grid=(n // bn,),
        in_specs=[
            pl.BlockSpec((bn, q0.shape[1]), lambda i: (i, 0)),
            pl.BlockSpec((bn, q0.shape[1]), lambda i: (i, 0)),
            pl.BlockSpec(wlt.shape, lambda i: (0, 0)),
            pl.BlockSpec(blrow.shape, lambda i: (0, 0)),
        ],
        out_specs=pl.BlockSpec((bn, n_cls), lambda i: (i, 0)),
        out_shape=jax.ShapeDtypeStruct((n, n_cls), jnp.float32),
    )(q0, q1, wlt, blrow)


def _sc_agg(xt, src, dst, w, n_acc):
    n, d = xt.shape
    e = src.shape[0]
    nw = NC * NS
    chunk = 80
    per_w = e // nw
    n_chunks = per_w // chunk
    assert per_w * nw == e and n_chunks * chunk == per_w and n_acc % NS == 0
    rows_t = n_acc // NS
    zrows = 128
    assert rows_t % zrows == 0
    mesh = plsc.VectorSubcoreMesh(core_axis_name="c", subcore_axis_name="s",
                                  num_cores=NC, num_subcores=NS)

    @functools.partial(
        pl.kernel,
        out_type=jax.ShapeDtypeStruct((NC, n_acc, d), jnp.float32),
        mesh=mesh,
        scratch_types=[
            pltpu.VMEM_SHARED((n_acc, d), jnp.float32),
            pltpu.VMEM((chunk,), jnp.int32),
            pltpu.VMEM((chunk,), jnp.int32),
            pltpu.VMEM((chunk, LANES), jnp.float32),
            pltpu.VMEM((chunk, d), jnp.float32),
            pltpu.VMEM((zrows, d), jnp.float32),
            pltpu.SemaphoreType.DMA,
        ],
    )
    def agg(xt_hbm, src_hbm, dst_hbm, w_hbm, out_hbm,
            acc_sh, src_v, dst_v, w_v, rows_v, zbuf, sem):
        cid = lax.axis_index("c")
        sid = lax.axis_index("s")
        wid = sid * NC + cid

        def zrow(i, _):
            for k in range(d // LANES):
                zbuf[i, pl.ds(k * LANES, LANES)] = jnp.zeros((LANES,), jnp.float32)
            return 0
        lax.fori_loop(0, zrows, zrow, 0)
        for t in range(rows_t // zrows):
            pltpu.sync_copy(zbuf, acc_sh.at[pl.ds(sid * rows_t + t * zrows, zrows)])
        plsc.subcore_barrier()

        base = wid * per_w

        def chunk_body(j, _):
            off = base + j * chunk
            pltpu.sync_copy(src_hbm.at[pl.ds(off, chunk)], src_v)
            pltpu.sync_copy(dst_hbm.at[pl.ds(off, chunk)], dst_v)
            pltpu.sync_copy(w_hbm.at[pl.ds(off, chunk)], w_v)
            pltpu.async_copy(xt_hbm.at[src_v], rows_v, sem).wait()

            def row_body(i, _):
                wvec = w_v[i, :]
                for k in range(d // LANES):
                    sl = pl.ds(k * LANES, LANES)
                    rows_v[i, sl] = rows_v[i, sl] * wvec
                return 0
            lax.fori_loop(0, chunk, row_body, 0)
            pltpu.sync_copy(rows_v, acc_sh.at[dst_v], add=True)
            return 0
        lax.fori_loop(0, n_chunks, chunk_body, 0)
        plsc.subcore_barrier()

        for t in range(rows_t // zrows):
            r0 = sid * rows_t + t * zrows
            pltpu.sync_copy(acc_sh.at[pl.ds(r0, zrows)], zbuf)
            pltpu.sync_copy(zbuf, out_hbm.at[cid, pl.ds(r0, zrows)])

    return agg(xt, src, dst, w)


def kernel(x, edge_index, edge_weight, W1, b1, W2, b2, W_lin, b_lin):
    n, d_in = x.shape
    d_h = W1.shape[0]
    d_out = W2.shape[0]
    n_cls = W_lin.shape[0]
    d1 = 128
    d2 = 128

    src = edge_index[0]
    dst = edge_index[1]

    w1t = jnp.zeros((d_in, d1), jnp.float32).at[:, :d_h].set(W1.T)
    b1row = jnp.zeros((1, d1), jnp.float32).at[0, :d_h].set(b1)
    w2t = jnp.zeros((d1, d2), jnp.float32).at[:d_h, :d_out].set(W2.T)
    b2row = jnp.zeros((1, d2), jnp.float32).at[0, :d_out].set(b2)
    wlt = jnp.zeros((d2, 128), jnp.float32).at[:d_out, :n_cls].set(W_lin.T)

    blrow = jnp.zeros((1, 128), jnp.float32).at[0, :n_cls].set(b_lin)

    n_acc = 10240
    wb = jnp.broadcast_to(edge_weight[:, None], (edge_weight.shape[0], LANES))
    xt1 = _stage_a(x, w1t, b1row, d1)
    p = _sc_agg(xt1, src, dst, wb, n_acc)
    xt2 = _stage_b(p[0], p[1], w2t, b2row, d2, bn=1024)
    q = _sc_agg(xt2, src, dst, wb, n_acc)
    return _stage_c(q[0], q[1], wlt, blrow, n_cls, bn=1024)[:n]


# speedup vs baseline: 5.1383x; 1.4796x over previous
import functools

import jax
import jax.numpy as jnp
from jax import lax
from jax.experimental import pallas as pl
from jax.experimental.pallas import tpu as pltpu
from jax.experimental.pallas import tpu_sc as plsc

MIN_NORM = 1e-15
EPS = 1e-7
MAX_NORM = 1e6

NC = 2
NS = 16
LANES = 16


def _col0_mask(v):
    col = lax.broadcasted_iota(jnp.int32, v.shape, 1)
    return col == 0


def _mask0(v):
    return jnp.where(_col0_mask(v), 0.0, v)


def _sinh(t):
    return 0.5 * (jnp.exp(t) - jnp.exp(-t))


def _arccosh(z):
    inv = 1.0 / z
    return jnp.log(z) + jnp.log1p(jnp.sqrt(jnp.maximum(1.0 - inv * inv, 0.0)))


def _rownorm(y):
    return jnp.maximum(jnp.sqrt(jnp.sum(y * y, axis=1, keepdims=True)), MIN_NORM)


def _proj(x):
    y = _mask0(x)
    x0 = jnp.sqrt(jnp.maximum(1.0 + jnp.sum(y * y, axis=1, keepdims=True), MIN_NORM))
    return jnp.where(_col0_mask(x), x0, x)


def _expmap0_proj(u):
    y = _mask0(u)
    yn = _rownorm(y)
    rest = (_sinh(yn) / yn) * y
    return _proj(rest)


def _logmap0(x):
    y = _mask0(x)
    yn = _rownorm(y)
    theta = jnp.maximum(x[:, 0:1], 1.0 + EPS)
    return (_arccosh(theta) / yn) * y


def _mobius_add_bias(x, hyp_bias):
    u = _logmap0(hyp_bias)
    x0 = x[:, 0:1]
    y = _mask0(x)
    yn = _rownorm(y)
    yu = y / yn
    v = jnp.where(_col0_mask(x), -yn, (1.0 - x0) * yu)
    alpha = jnp.sum(yu * _mask0(u), axis=1, keepdims=True)
    res = u - alpha * v
    ux = jnp.sum(y * _mask0(res), axis=1, keepdims=True)
    pt = jnp.where(_col0_mask(x), ux / jnp.maximum(x0, EPS), res)
    mink = jnp.sum(pt * pt, axis=1, keepdims=True) - 2.0 * pt[:, 0:1] * pt[:, 0:1]
    normu = jnp.minimum(jnp.sqrt(jnp.maximum(mink, EPS)), MAX_NORM)
    theta = jnp.maximum(normu, MIN_NORM)
    ch = 0.5 * (jnp.exp(theta) + jnp.exp(-theta))
    r = ch * x + (_sinh(theta) / theta) * pt
    return _proj(r)


def _hyp_bias(b_row):
    return _expmap0_proj(_mask0(b_row))


def _stage_a_body(x_ref, w1t_ref, b1_ref, o_ref):
    xb = x_ref[...]
    y = _mask0(xb)
    yn = _rownorm(y)
    rest = (_sinh(yn) / yn) * y
    x0 = jnp.sqrt(jnp.maximum(1.0 + jnp.sum(rest * rest, axis=1, keepdims=True),
                              MIN_NORM))
    yn2 = _rownorm(rest)
    theta = jnp.maximum(x0, 1.0 + EPS)
    u = (_arccosh(theta) / yn2) * rest
    mu = jnp.dot(u, w1t_ref[...], preferred_element_type=jnp.float32)
    res = _expmap0_proj(mu)
    res = _mobius_add_bias(res, _hyp_bias(b1_ref[...]))
    o_ref[...] = _logmap0(res)


def _stage_a(x, w1t, b1row, d1, bn=400):
    n = x.shape[0]
    return pl.pallas_call(
        _stage_a_body,
        grid=(n // bn,),
        in_specs=[
            pl.BlockSpec((bn, x.shape[1]), lambda i: (i, 0)),
            pl.BlockSpec(w1t.shape, lambda i: (0, 0)),
            pl.BlockSpec(b1row.shape, lambda i: (0, 0)),
        ],
        out_specs=pl.BlockSpec((bn, d1), lambda i: (i, 0)),
        out_shape=jax.ShapeDtypeStruct((n, d1), jnp.float32),
    )(x, w1t, b1row)


def _stage_b_body(p0_ref, p1_ref, w2t_ref, b2_ref, o_ref):
    support = p0_ref[...] + p1_ref[...]
    h = _expmap0_proj(support)
    xt = jax.nn.relu(_logmap0(h))
    h2 = _expmap0_proj(xt)
    u = _logmap0(h2)
    mu = jnp.dot(u, w2t_ref[...], preferred_element_type=jnp.float32)
    res = _expmap0_proj(mu)
    res = _mobius_add_bias(res, _hyp_bias(b2_ref[...]))
    o_ref[...] = _logmap0(res)


def _stage_b(p0, p1, w2t, b2row, d2, bn=1000):
    n = p0.shape[0]
    return pl.pallas_call(
        _stage_b_body,
        grid=(n // bn,),
        in_specs=[
            pl.BlockSpec((bn, p0.shape[1]), lambda i: (i, 0)),
            pl.BlockSpec((bn, p0.shape[1]), lambda i: (i, 0)),
            pl.BlockSpec(w2t.shape, lambda i: (0, 0)),
            pl.BlockSpec(b2row.shape, lambda i: (0, 0)),
        ],
        out_specs=pl.BlockSpec((bn, d2), lambda i: (i, 0)),
        out_shape=jax.ShapeDtypeStruct((n, d2), jnp.float32),
    )(p0, p1, w2t, b2row)


def _stage_c_body(q0_ref, q1_ref, wlt_ref, bl_ref, o_ref, *, n_cls):
    support = q0_ref[...] + q1_ref[...]
    h = _expmap0_proj(support)
    xt = jax.nn.relu(_logmap0(h))
    h2 = _expmap0_proj(xt)
    ht = _logmap0(h2)
    logits = jnp.dot(ht, wlt_ref[...], preferred_element_type=jnp.float32)
    logits = jax.nn.relu(logits + bl_ref[...])
    col = lax.broadcasted_iota(jnp.int32, logits.shape, 1)
    masked = jnp.where(col < n_cls, logits, -jnp.inf)
    m = jnp.max(masked, axis=1, keepdims=True)
    z = masked - m
    lse = jnp.log(jnp.sum(jnp.where(col < n_cls, jnp.exp(z), 0.0),
                          axis=1, keepdims=True))
    o_ref[...] = (z - lse)[:, :n_cls]


def _stage_c(q0, q1, wlt, blrow, n_cls, bn=1000):
    n = q0.shape[0]
    return pl.pallas_call(
        functools.partial(_stage_c_body, n_cls=n_cls),
        grid=(n // bn,),
        in_specs=[
            pl.BlockSpec((bn, q0.shape[1]), lambda i: (i, 0)),
            pl.BlockSpec((bn, q0.shape[1]), lambda i: (i, 0)),
            pl.BlockSpec(wlt.shape, lambda i: (0, 0)),
            pl.BlockSpec(blrow.shape, lambda i: (0, 0)),
        ],
        out_specs=pl.BlockSpec((bn, n_cls), lambda i: (i, 0)),
        out_shape=jax.ShapeDtypeStruct((n, n_cls), jnp.float32),
    )(q0, q1, wlt, blrow)


def _sc_agg(xt, src, dst, w, n_acc):
    n, d = xt.shape
    e = src.shape[0]
    nw = NC * NS
    chunk = 40
    n_glob = e // chunk
    assert n_glob * chunk == e and n_acc % NS == 0
    base_cnt = n_glob // nw
    extra = n_glob - base_cnt * nw
    nch = base_cnt + (1 if extra else 0)
    nch += (1 - nch) % 3
    assert nch >= 4 and (nch - 4) % 3 == 0
    rows_t = n_acc // NS
    zrows = chunk
    assert rows_t % zrows == 0
    mesh = plsc.VectorSubcoreMesh(core_axis_name="c", subcore_axis_name="s",
                                  num_cores=NC, num_subcores=NS)

    @functools.partial(
        pl.kernel,
        out_type=jax.ShapeDtypeStruct((NC, n_acc, d), jnp.float32),
        mesh=mesh,
        scratch_types=[
            pltpu.VMEM_SHARED((n_acc, d), jnp.float32),
            [pltpu.VMEM((chunk,), jnp.int32)] * 3,
            [pltpu.VMEM((chunk,), jnp.int32)] * 3,
            [pltpu.VMEM((chunk, LANES), jnp.float32)] * 3,
            [pltpu.VMEM((chunk, d), jnp.float32)] * 3,
            [pltpu.SemaphoreType.DMA] * 3,
            [pltpu.SemaphoreType.DMA] * 3,
            [pltpu.SemaphoreType.DMA] * 3,
        ],
    )
    def agg(xt_hbm, src_hbm, dst_hbm, w_hbm, out_hbm,
            acc_sh, srcv, dstv, wv, rows, gsem, isem, ssem):
        zbuf = rows[0]
        cid = lax.axis_index("c")
        sid = lax.axis_index("s")
        wid = sid * NC + cid
        n_j = base_cnt + jnp.where(wid < extra, 1, 0)

        def zrow(i, _):
            for k in range(d // LANES):
                zbuf[i, pl.ds(k * LANES, LANES)] = jnp.zeros((LANES,), jnp.float32)
            return 0
        lax.fori_loop(0, zrows, zrow, 0)
        for t in range(rows_t // zrows):
            pltpu.sync_copy(zbuf, acc_sh.at[pl.ds(sid * rows_t + t * zrows, zrows)])
        plsc.subcore_barrier()

        def off_of(j):
            jj = jnp.where(j < n_j, j, 0)
            return (wid + nw * jj) * chunk

        def issue_idx(j, b):
            off = off_of(j)
            pltpu.async_copy(src_hbm.at[pl.ds(off, chunk)], srcv[b], isem[b])
            pltpu.async_copy(dst_hbm.at[pl.ds(off, chunk)], dstv[b], isem[b])
            pltpu.async_copy(w_hbm.at[pl.ds(off, chunk)], wv[b], isem[b])

        def wait_idx(j, b):
            off = off_of(j)
            pltpu.make_async_copy(src_hbm.at[pl.ds(off, chunk)], srcv[b], isem[b]).wait()
            pltpu.make_async_copy(dst_hbm.at[pl.ds(off, chunk)], dstv[b], isem[b]).wait()
            pltpu.make_async_copy(w_hbm.at[pl.ds(off, chunk)], wv[b], isem[b]).wait()

        def issue_gather(b):
            pltpu.async_copy(xt_hbm.at[srcv[b]], rows[b], gsem[b])

        def wait_gather(b):
            pltpu.make_async_copy(xt_hbm.at[srcv[b]], rows[b], gsem[b]).wait()

        def issue_scatter(b):
            pltpu.async_copy(rows[b], acc_sh.at[dstv[b]], ssem[b], add=True)

        def wait_scatter(b):
            pltpu.make_async_copy(rows[b], acc_sh.at[dstv[b]], ssem[b]).wait()

        def multiply(b):
            def rb(i, _):
                for u in range(2):
                    r = i * 2 + u
                    wvec = wv[b][r, :]
                    for k in range(d // LANES):
                        sl = pl.ds(k * LANES, LANES)
                        rows[b][r, sl] = rows[b][r, sl] * wvec
                return 0
            lax.fori_loop(0, chunk // 2, rb, 0)

        def compute(j, b):
            wait_gather(b)

            @pl.when(j >= n_j)
            def _():
                for k in range(chunk):
                    wv[b][k, :] = jnp.zeros((LANES,), jnp.float32)
            multiply(b)
            issue_scatter(b)

        issue_idx(0, 0)
        issue_idx(1, 1)
        issue_idx(2, 2)
        wait_idx(0, 0)
        issue_gather(0)
        wait_idx(1, 1)
        issue_gather(1)
        compute(0, 0)
        wait_idx(2, 2)
        issue_gather(2)
        compute(1, 1)
        wait_scatter(0)
        issue_idx(3, 0)

        def steady(g, _):
            j0 = 2 + 3 * g
            for t in range(3):
                j = j0 + t
                b, b1, b2 = (2 + t) % 3, t % 3, (t + 1) % 3
                wait_idx(j + 1, b1)
                issue_gather(b1)
                compute(j, b)
                wait_scatter(b2)
                issue_idx(j + 2, b2)
            return 0
        lax.fori_loop(0, (nch - 4) // 3, steady, 0)
        bj = (nch - 2) % 3
        wait_idx(nch - 1, (nch - 1) % 3)
        issue_gather((nch - 1) % 3)
        compute(nch - 2, bj)
        compute(nch - 1, (nch - 1) % 3)
        wait_scatter((nch - 3) % 3)
        wait_scatter((nch - 2) % 3)
        wait_scatter((nch - 1) % 3)
        plsc.subcore_barrier()

        for t in range(rows_t // zrows):
            r0 = sid * rows_t + t * zrows
            pltpu.sync_copy(acc_sh.at[pl.ds(r0, zrows)], zbuf)
            pltpu.sync_copy(zbuf, out_hbm.at[cid, pl.ds(r0, zrows)])

    return agg(xt, src, dst, w)


def kernel(x, edge_index, edge_weight, W1, b1, W2, b2, W_lin, b_lin):
    n, d_in = x.shape
    d_h = W1.shape[0]
    d_out = W2.shape[0]
    n_cls = W_lin.shape[0]
    d1 = 128
    d2 = 128

    src = edge_index[0]
    dst = edge_index[1]

    w1t = jnp.zeros((d_in, d1), jnp.float32).at[:, :d_h].set(W1.T)
    b1row = jnp.zeros((1, d1), jnp.float32).at[0, :d_h].set(b1)
    w2t = jnp.zeros((d1, d2), jnp.float32).at[:d_h, :d_out].set(W2.T)
    b2row = jnp.zeros((1, d2), jnp.float32).at[0, :d_out].set(b2)
    wlt = jnp.zeros((d2, 128), jnp.float32).at[:d_out, :n_cls].set(W_lin.T)

    blrow = jnp.zeros((1, 128), jnp.float32).at[0, :n_cls].set(b_lin)

    n_acc = 10240
    wb = jnp.broadcast_to(edge_weight[:, None], (edge_weight.shape[0], LANES))
    xt1 = _stage_a(x, w1t, b1row, d1)
    p = _sc_agg(xt1, src, dst, wb, n_acc)
    xt2 = _stage_b(p[0], p[1], w2t, b2row, d2, bn=1024)
    q = _sc_agg(xt2, src, dst, wb, n_acc)
    return _stage_c(q[0], q[1], wlt, blrow, n_cls, bn=1024)[:n]


# speedup vs baseline: 6.0690x; 1.1811x over previous
import functools

import jax
import jax.numpy as jnp
from jax import lax
from jax.experimental import pallas as pl
from jax.experimental.pallas import tpu as pltpu
from jax.experimental.pallas import tpu_sc as plsc

MIN_NORM = 1e-15
EPS = 1e-7
MAX_NORM = 1e6

NC = 2
NS = 16
LANES = 16


def _col0_mask(v):
    col = lax.broadcasted_iota(jnp.int32, v.shape, 1)
    return col == 0


def _mask0(v):
    return jnp.where(_col0_mask(v), 0.0, v)


def _sinh(t):
    return 0.5 * (jnp.exp(t) - jnp.exp(-t))


def _arccosh(z):
    inv = 1.0 / z
    return jnp.log(z) + jnp.log1p(jnp.sqrt(jnp.maximum(1.0 - inv * inv, 0.0)))


def _rownorm(y):
    return jnp.maximum(jnp.sqrt(jnp.sum(y * y, axis=1, keepdims=True)), MIN_NORM)


def _proj(x):
    y = _mask0(x)
    x0 = jnp.sqrt(jnp.maximum(1.0 + jnp.sum(y * y, axis=1, keepdims=True), MIN_NORM))
    return jnp.where(_col0_mask(x), x0, x)


def _expmap0_proj(u):
    y = _mask0(u)
    yn = _rownorm(y)
    rest = (_sinh(yn) / yn) * y
    return _proj(rest)


def _logmap0(x):
    y = _mask0(x)
    yn = _rownorm(y)
    theta = jnp.maximum(x[:, 0:1], 1.0 + EPS)
    return (_arccosh(theta) / yn) * y


def _mobius_add_bias(x, hyp_bias):
    u = _logmap0(hyp_bias)
    x0 = x[:, 0:1]
    y = _mask0(x)
    yn = _rownorm(y)
    yu = y / yn
    v = jnp.where(_col0_mask(x), -yn, (1.0 - x0) * yu)
    alpha = jnp.sum(yu * _mask0(u), axis=1, keepdims=True)
    res = u - alpha * v
    ux = jnp.sum(y * _mask0(res), axis=1, keepdims=True)
    pt = jnp.where(_col0_mask(x), ux / jnp.maximum(x0, EPS), res)
    mink = jnp.sum(pt * pt, axis=1, keepdims=True) - 2.0 * pt[:, 0:1] * pt[:, 0:1]
    normu = jnp.minimum(jnp.sqrt(jnp.maximum(mink, EPS)), MAX_NORM)
    theta = jnp.maximum(normu, MIN_NORM)
    ch = 0.5 * (jnp.exp(theta) + jnp.exp(-theta))
    r = ch * x + (_sinh(theta) / theta) * pt
    return _proj(r)


def _hyp_bias(b_row):
    return _expmap0_proj(_mask0(b_row))


def _stage_a_body(x_ref, w1t_ref, b1_ref, o_ref):
    xb = x_ref[...]
    y = _mask0(xb)
    yn = _rownorm(y)
    rest = (_sinh(yn) / yn) * y
    x0 = jnp.sqrt(jnp.maximum(1.0 + jnp.sum(rest * rest, axis=1, keepdims=True),
                              MIN_NORM))
    yn2 = _rownorm(rest)
    theta = jnp.maximum(x0, 1.0 + EPS)
    u = (_arccosh(theta) / yn2) * rest
    mu = jnp.dot(u, w1t_ref[...], preferred_element_type=jnp.float32)
    res = _expmap0_proj(mu)
    res = _mobius_add_bias(res, _hyp_bias(b1_ref[...]))
    o_ref[...] = _logmap0(res)


def _stage_a(x, w1t, b1row, d1, bn=400):
    n = x.shape[0]
    return pl.pallas_call(
        _stage_a_body,
        grid=(n // bn,),
        in_specs=[
            pl.BlockSpec((bn, x.shape[1]), lambda i: (i, 0)),
            pl.BlockSpec(w1t.shape, lambda i: (0, 0)),
            pl.BlockSpec(b1row.shape, lambda i: (0, 0)),
        ],
        out_specs=pl.BlockSpec((bn, d1), lambda i: (i, 0)),
        out_shape=jax.ShapeDtypeStruct((n, d1), jnp.float32),
    )(x, w1t, b1row)


def _stage_b_body(p0_ref, p1_ref, w2t_ref, b2_ref, o_ref):
    support = p0_ref[...] + p1_ref[...]
    h = _expmap0_proj(support)
    xt = jax.nn.relu(_logmap0(h))
    h2 = _expmap0_proj(xt)
    u = _logmap0(h2)
    mu = jnp.dot(u, w2t_ref[...], preferred_element_type=jnp.float32)
    res = _expmap0_proj(mu)
    res = _mobius_add_bias(res, _hyp_bias(b2_ref[...]))
    o_ref[...] = _logmap0(res)


def _stage_b(p0, p1, w2t, b2row, d2, bn=1000):
    n = p0.shape[0]
    return pl.pallas_call(
        _stage_b_body,
        grid=(n // bn,),
        in_specs=[
            pl.BlockSpec((bn, p0.shape[1]), lambda i: (i, 0)),
            pl.BlockSpec((bn, p0.shape[1]), lambda i: (i, 0)),
            pl.BlockSpec(w2t.shape, lambda i: (0, 0)),
            pl.BlockSpec(b2row.shape, lambda i: (0, 0)),
        ],
        out_specs=pl.BlockSpec((bn, d2), lambda i: (i, 0)),
        out_shape=jax.ShapeDtypeStruct((n, d2), jnp.float32),
    )(p0, p1, w2t, b2row)


def _stage_c_body(q0_ref, q1_ref, wlt_ref, bl_ref, o_ref, *, n_cls):
    support = q0_ref[...] + q1_ref[...]
    h = _expmap0_proj(support)
    xt = jax.nn.relu(_logmap0(h))
    h2 = _expmap0_proj(xt)
    ht = _logmap0(h2)
    logits = jnp.dot(ht, wlt_ref[...], preferred_element_type=jnp.float32)
    logits = jax.nn.relu(logits + bl_ref[...])
    col = lax.broadcasted_iota(jnp.int32, logits.shape, 1)
    masked = jnp.where(col < n_cls, logits, -jnp.inf)
    m = jnp.max(masked, axis=1, keepdims=True)
    z = masked - m
    lse = jnp.log(jnp.sum(jnp.where(col < n_cls, jnp.exp(z), 0.0),
                          axis=1, keepdims=True))
    o_ref[...] = (z - lse)[:, :n_cls]


def _stage_c(q0, q1, wlt, blrow, n_cls, bn=1000):
    n = q0.shape[0]
    return pl.pallas_call(
        functools.partial(_stage_c_body, n_cls=n_cls),
        grid=(n // bn,),
        in_specs=[
            pl.BlockSpec((bn, q0.shape[1]), lambda i: (i, 0)),
            pl.BlockSpec((bn, q0.shape[1]), lambda i: (i, 0)),
            pl.BlockSpec(wlt.shape, lambda i: (0, 0)),
            pl.BlockSpec(blrow.shape, lambda i: (0, 0)),
        ],
        out_specs=pl.BlockSpec((bn, n_cls), lambda i: (i, 0)),
        out_shape=jax.ShapeDtypeStruct((n, n_cls), jnp.float32),
    )(q0, q1, wlt, blrow)


def _sc_agg(xt, src, dst, w, n_acc):
    n, d = xt.shape
    e = src.shape[0]
    nw = NC * NS
    chunk = 64
    n_glob = e // chunk
    assert n_glob * chunk == e and n_acc % NS == 0
    base_cnt = n_glob // nw
    extra = n_glob - base_cnt * nw
    nch = base_cnt + (1 if extra else 0)
    nch += (1 - nch) % 3
    assert nch >= 4 and (nch - 4) % 3 == 0
    rows_t = n_acc // NS
    zrows = chunk
    segs = [(r0, min(zrows, rows_t - r0)) for r0 in range(0, rows_t, zrows)]
    assert all(sz % 8 == 0 for _, sz in segs)
    mesh = plsc.VectorSubcoreMesh(core_axis_name="c", subcore_axis_name="s",
                                  num_cores=NC, num_subcores=NS)

    @functools.partial(
        pl.kernel,
        out_type=jax.ShapeDtypeStruct((NC, n_acc, d), jnp.float32),
        mesh=mesh,
        scratch_types=[
            pltpu.VMEM_SHARED((n_acc, d), jnp.float32),
            [pltpu.VMEM((chunk,), jnp.int32)] * 3,
            [pltpu.VMEM((chunk,), jnp.int32)] * 3,
            [pltpu.VMEM((chunk, LANES), jnp.float32)] * 3,
            [pltpu.VMEM((chunk, d), jnp.float32)] * 3,
            [pltpu.SemaphoreType.DMA] * 3,
            [pltpu.SemaphoreType.DMA] * 3,
            [pltpu.SemaphoreType.DMA] * 3,
        ],
    )
    def agg(xt_hbm, src_hbm, dst_hbm, w_hbm, out_hbm,
            acc_sh, srcv, dstv, wv, rows, gsem, isem, ssem):
        zbuf = rows[0]
        cid = lax.axis_index("c")
        sid = lax.axis_index("s")
        wid = sid * NC + cid
        n_j = base_cnt + jnp.where(wid < extra, 1, 0)

        def zrow(i, _):
            for k in range(d // LANES):
                zbuf[i, pl.ds(k * LANES, LANES)] = jnp.zeros((LANES,), jnp.float32)
            return 0
        lax.fori_loop(0, zrows, zrow, 0)
        for r0, sz in segs:
            pltpu.sync_copy(zbuf.at[pl.ds(0, sz)],
                            acc_sh.at[pl.ds(sid * rows_t + r0, sz)])
        plsc.subcore_barrier()

        def off_of(j):
            jj = jnp.where(j < n_j, j, 0)
            return (wid + nw * jj) * chunk

        def issue_idx(j, b):
            off = off_of(j)
            pltpu.async_copy(src_hbm.at[pl.ds(off, chunk)], srcv[b], isem[b])
            pltpu.async_copy(dst_hbm.at[pl.ds(off, chunk)], dstv[b], isem[b])
            pltpu.async_copy(w_hbm.at[pl.ds(off, chunk)], wv[b], isem[b])

        def wait_idx(j, b):
            off = off_of(j)
            pltpu.make_async_copy(src_hbm.at[pl.ds(off, chunk)], srcv[b], isem[b]).wait()
            pltpu.make_async_copy(dst_hbm.at[pl.ds(off, chunk)], dstv[b], isem[b]).wait()
            pltpu.make_async_copy(w_hbm.at[pl.ds(off, chunk)], wv[b], isem[b]).wait()

        def issue_gather(b):
            pltpu.async_copy(xt_hbm.at[srcv[b]], rows[b], gsem[b])

        def wait_gather(b):
            pltpu.make_async_copy(xt_hbm.at[srcv[b]], rows[b], gsem[b]).wait()

        def issue_scatter(b):
            pltpu.async_copy(rows[b], acc_sh.at[dstv[b]], ssem[b], add=True)

        def wait_scatter(b):
            pltpu.make_async_copy(rows[b], acc_sh.at[dstv[b]], ssem[b]).wait()

        def multiply(b):
            def rb(i, _):
                for u in range(2):
                    r = i * 2 + u
                    wvec = wv[b][r, :]
                    for k in range(d // LANES):
                        sl = pl.ds(k * LANES, LANES)
                        rows[b][r, sl] = rows[b][r, sl] * wvec
                return 0
            lax.fori_loop(0, chunk // 2, rb, 0)

        def compute(j, b):
            wait_gather(b)

            @pl.when(j >= n_j)
            def _():
                for k in range(chunk):
                    wv[b][k, :] = jnp.zeros((LANES,), jnp.float32)
            multiply(b)
            issue_scatter(b)

        issue_idx(0, 0)
        issue_idx(1, 1)
        issue_idx(2, 2)
        wait_idx(0, 0)
        issue_gather(0)
        wait_idx(1, 1)
        issue_gather(1)
        compute(0, 0)
        wait_idx(2, 2)
        issue_gather(2)
        compute(1, 1)
        wait_scatter(0)
        issue_idx(3, 0)

        def steady(g, _):
            j0 = 2 + 3 * g
            for t in range(3):
                j = j0 + t
                b, b1, b2 = (2 + t) % 3, t % 3, (t + 1) % 3
                wait_idx(j + 1, b1)
                issue_gather(b1)
                compute(j, b)
                wait_scatter(b2)
                issue_idx(j + 2, b2)
            return 0
        lax.fori_loop(0, (nch - 4) // 3, steady, 0)
        bj = (nch - 2) % 3
        wait_idx(nch - 1, (nch - 1) % 3)
        issue_gather((nch - 1) % 3)
        compute(nch - 2, bj)
        compute(nch - 1, (nch - 1) % 3)
        wait_scatter((nch - 3) % 3)
        wait_scatter((nch - 2) % 3)
        wait_scatter((nch - 1) % 3)
        plsc.subcore_barrier()

        for r0, sz in segs:
            a0 = sid * rows_t + r0
            pltpu.sync_copy(acc_sh.at[pl.ds(a0, sz)], zbuf.at[pl.ds(0, sz)])
            pltpu.sync_copy(zbuf.at[pl.ds(0, sz)], out_hbm.at[cid, pl.ds(a0, sz)])

    return agg(xt, src, dst, w)


def kernel(x, edge_index, edge_weight, W1, b1, W2, b2, W_lin, b_lin):
    n, d_in = x.shape
    d_h = W1.shape[0]
    d_out = W2.shape[0]
    n_cls = W_lin.shape[0]
    d1 = 128
    d2 = 128

    src = edge_index[0]
    dst = edge_index[1]

    w1t = jnp.zeros((d_in, d1), jnp.float32).at[:, :d_h].set(W1.T)
    b1row = jnp.zeros((1, d1), jnp.float32).at[0, :d_h].set(b1)
    w2t = jnp.zeros((d1, d2), jnp.float32).at[:d_h, :d_out].set(W2.T)
    b2row = jnp.zeros((1, d2), jnp.float32).at[0, :d_out].set(b2)
    wlt = jnp.zeros((d2, 128), jnp.float32).at[:d_out, :n_cls].set(W_lin.T)

    blrow = jnp.zeros((1, 128), jnp.float32).at[0, :n_cls].set(b_lin)

    n_acc = 10112
    wb = jnp.broadcast_to(edge_weight[:, None], (edge_weight.shape[0], LANES))
    xt1 = _stage_a(x, w1t, b1row, d1)
    p = _sc_agg(xt1, src, dst, wb, n_acc)
    xt2 = _stage_b(p[0], p[1], w2t, b2row, d2, bn=1264)
    q = _sc_agg(xt2, src, dst, wb, n_acc)
    return _stage_c(q[0], q[1], wlt, blrow, n_cls, bn=1264)[:n]


# speedup vs baseline: 6.0719x; 1.0005x over previous
import functools

import jax
import jax.numpy as jnp
from jax import lax
from jax.experimental import pallas as pl
from jax.experimental.pallas import tpu as pltpu
from jax.experimental.pallas import tpu_sc as plsc

MIN_NORM = 1e-15
EPS = 1e-7
MAX_NORM = 1e6

NC = 2
NS = 16
LANES = 16


def _col0_mask(v):
    col = lax.broadcasted_iota(jnp.int32, v.shape, 1)
    return col == 0


def _mask0(v):
    return jnp.where(_col0_mask(v), 0.0, v)


def _sinh(t):
    return 0.5 * (jnp.exp(t) - jnp.exp(-t))


def _arccosh(z):
    inv = 1.0 / z
    return jnp.log(z) + jnp.log1p(jnp.sqrt(jnp.maximum(1.0 - inv * inv, 0.0)))


def _rownorm(y):
    return jnp.maximum(jnp.sqrt(jnp.sum(y * y, axis=1, keepdims=True)), MIN_NORM)


def _proj(x):
    y = _mask0(x)
    x0 = jnp.sqrt(jnp.maximum(1.0 + jnp.sum(y * y, axis=1, keepdims=True), MIN_NORM))
    return jnp.where(_col0_mask(x), x0, x)


def _expmap0_proj(u):
    y = _mask0(u)
    yn = _rownorm(y)
    rest = (_sinh(yn) / yn) * y
    return _proj(rest)


def _logmap0(x):
    y = _mask0(x)
    yn = _rownorm(y)
    theta = jnp.maximum(x[:, 0:1], 1.0 + EPS)
    return (_arccosh(theta) / yn) * y


def _mobius_add_bias(x, hyp_bias):
    u = _logmap0(hyp_bias)
    x0 = x[:, 0:1]
    y = _mask0(x)
    yn = _rownorm(y)
    yu = y / yn
    v = jnp.where(_col0_mask(x), -yn, (1.0 - x0) * yu)
    alpha = jnp.sum(yu * _mask0(u), axis=1, keepdims=True)
    res = u - alpha * v
    ux = jnp.sum(y * _mask0(res), axis=1, keepdims=True)
    pt = jnp.where(_col0_mask(x), ux / jnp.maximum(x0, EPS), res)
    mink = jnp.sum(pt * pt, axis=1, keepdims=True) - 2.0 * pt[:, 0:1] * pt[:, 0:1]
    normu = jnp.minimum(jnp.sqrt(jnp.maximum(mink, EPS)), MAX_NORM)
    theta = jnp.maximum(normu, MIN_NORM)
    ch = 0.5 * (jnp.exp(theta) + jnp.exp(-theta))
    r = ch * x + (_sinh(theta) / theta) * pt
    return _proj(r)


def _hyp_bias(b_row):
    return _expmap0_proj(_mask0(b_row))


def _stage_a_body(x_ref, w1t_ref, b1_ref, o_ref):
    xb = x_ref[...]
    y = _mask0(xb)
    yn = _rownorm(y)
    rest = (_sinh(yn) / yn) * y
    x0 = jnp.sqrt(jnp.maximum(1.0 + jnp.sum(rest * rest, axis=1, keepdims=True),
                              MIN_NORM))
    yn2 = _rownorm(rest)
    theta = jnp.maximum(x0, 1.0 + EPS)
    u = (_arccosh(theta) / yn2) * rest
    mu = jnp.dot(u, w1t_ref[...], preferred_element_type=jnp.float32)
    res = _expmap0_proj(mu)
    res = _mobius_add_bias(res, _hyp_bias(b1_ref[...]))
    o_ref[...] = _logmap0(res)


def _stage_a(x, w1t, b1row, d1, bn=400):
    n = x.shape[0]
    return pl.pallas_call(
        _stage_a_body,
        grid=(n // bn,),
        in_specs=[
            pl.BlockSpec((bn, x.shape[1]), lambda i: (i, 0)),
            pl.BlockSpec(w1t.shape, lambda i: (0, 0)),
            pl.BlockSpec(b1row.shape, lambda i: (0, 0)),
        ],
        out_specs=pl.BlockSpec((bn, d1), lambda i: (i, 0)),
        out_shape=jax.ShapeDtypeStruct((n, d1), jnp.float32),
    )(x, w1t, b1row)


def _stage_b_body(p0_ref, p1_ref, w2t_ref, b2_ref, o_ref):
    support = p0_ref[...] + p1_ref[...]
    h = _expmap0_proj(support)
    xt = jax.nn.relu(_logmap0(h))
    h2 = _expmap0_proj(xt)
    u = _logmap0(h2)
    mu = jnp.dot(u, w2t_ref[...], preferred_element_type=jnp.float32)
    res = _expmap0_proj(mu)
    res = _mobius_add_bias(res, _hyp_bias(b2_ref[...]))
    o_ref[...] = _logmap0(res)


def _stage_b(p0, p1, w2t, b2row, d2, bn=1000):
    n = p0.shape[0]
    return pl.pallas_call(
        _stage_b_body,
        grid=(n // bn,),
        in_specs=[
            pl.BlockSpec((bn, p0.shape[1]), lambda i: (i, 0)),
            pl.BlockSpec((bn, p0.shape[1]), lambda i: (i, 0)),
            pl.BlockSpec(w2t.shape, lambda i: (0, 0)),
            pl.BlockSpec(b2row.shape, lambda i: (0, 0)),
        ],
        out_specs=pl.BlockSpec((bn, d2), lambda i: (i, 0)),
        out_shape=jax.ShapeDtypeStruct((n, d2), jnp.float32),
    )(p0, p1, w2t, b2row)


def _stage_c_body(q0_ref, q1_ref, wlt_ref, bl_ref, o_ref, *, n_cls):
    support = q0_ref[...] + q1_ref[...]
    h = _expmap0_proj(support)
    xt = jax.nn.relu(_logmap0(h))
    h2 = _expmap0_proj(xt)
    ht = _logmap0(h2)
    logits = jnp.dot(ht, wlt_ref[...], preferred_element_type=jnp.float32)
    logits = jax.nn.relu(logits + bl_ref[...])
    col = lax.broadcasted_iota(jnp.int32, logits.shape, 1)
    masked = jnp.where(col < n_cls, logits, -jnp.inf)
    m = jnp.max(masked, axis=1, keepdims=True)
    z = masked - m
    lse = jnp.log(jnp.sum(jnp.where(col < n_cls, jnp.exp(z), 0.0),
                          axis=1, keepdims=True))
    o_ref[...] = (z - lse)[:, :n_cls]


def _stage_c(q0, q1, wlt, blrow, n_cls, bn=1000):
    n = q0.shape[0]
    return pl.pallas_call(
        functools.partial(_stage_c_body, n_cls=n_cls),
        grid=(n // bn,),
        in_specs=[
            pl.BlockSpec((bn, q0.shape[1]), lambda i: (i, 0)),
            pl.BlockSpec((bn, q0.shape[1]), lambda i: (i, 0)),
            pl.BlockSpec(wlt.shape, lambda i: (0, 0)),
            pl.BlockSpec(blrow.shape, lambda i: (0, 0)),
        ],
        out_specs=pl.BlockSpec((bn, n_cls), lambda i: (i, 0)),
        out_shape=jax.ShapeDtypeStruct((n, n_cls), jnp.float32),
    )(q0, q1, wlt, blrow)


def _sc_agg(xt, src, dst, w, n_acc):
    n, d = xt.shape
    e = src.shape[0]
    nw = NC * NS
    chunk = 64
    n_glob = e // chunk
    assert n_glob * chunk == e and n_acc % NS == 0
    base_cnt = n_glob // nw
    extra = n_glob - base_cnt * nw
    nch = base_cnt + (1 if extra else 0)
    nch += (1 - nch) % 3
    assert nch >= 4 and (nch - 4) % 3 == 0
    rows_t = n_acc // NS
    zrows = chunk
    segs = [(r0, min(zrows, rows_t - r0)) for r0 in range(0, rows_t, zrows)]
    assert all(sz % 8 == 0 for _, sz in segs)
    mesh = plsc.VectorSubcoreMesh(core_axis_name="c", subcore_axis_name="s",
                                  num_cores=NC, num_subcores=NS)

    @functools.partial(
        pl.kernel,
        out_type=jax.ShapeDtypeStruct((NC, n_acc, d), jnp.float32),
        mesh=mesh,
        scratch_types=[
            pltpu.VMEM_SHARED((n_acc, d), jnp.float32),
            [pltpu.VMEM((chunk,), jnp.int32)] * 3,
            [pltpu.VMEM((chunk,), jnp.int32)] * 3,
            [pltpu.VMEM((chunk, LANES), jnp.float32)] * 3,
            [pltpu.VMEM((chunk, d), jnp.float32)] * 3,
            [pltpu.SemaphoreType.DMA] * 3,
            [pltpu.SemaphoreType.DMA] * 3,
            [pltpu.SemaphoreType.DMA] * 3,
        ],
    )
    def agg(xt_hbm, src_hbm, dst_hbm, w_hbm, out_hbm,
            acc_sh, srcv, dstv, wv, rows, gsem, isem, ssem):
        zbuf = rows[0]
        cid = lax.axis_index("c")
        sid = lax.axis_index("s")
        wid = sid * NC + cid
        n_j = base_cnt + jnp.where(wid < extra, 1, 0)

        def zrow(i, _):
            for k in range(d // LANES):
                zbuf[i, pl.ds(k * LANES, LANES)] = jnp.zeros((LANES,), jnp.float32)
            return 0
        lax.fori_loop(0, zrows, zrow, 0)
        for r0, sz in segs:
            pltpu.sync_copy(zbuf.at[pl.ds(0, sz)],
                            acc_sh.at[pl.ds(sid * rows_t + r0, sz)])
        plsc.subcore_barrier()

        def off_of(j):
            jj = jnp.where(j < n_j, j, 0)
            return (wid + nw * jj) * chunk

        def issue_idx(j, b):
            off = off_of(j)
            pltpu.async_copy(src_hbm.at[pl.ds(off, chunk)], srcv[b], isem[b])
            pltpu.async_copy(dst_hbm.at[pl.ds(off, chunk)], dstv[b], isem[b])
            pltpu.async_copy(w_hbm.at[pl.ds(off, chunk)], wv[b], isem[b])

        def wait_idx(j, b):
            off = off_of(j)
            pltpu.make_async_copy(src_hbm.at[pl.ds(off, chunk)], srcv[b], isem[b]).wait()
            pltpu.make_async_copy(dst_hbm.at[pl.ds(off, chunk)], dstv[b], isem[b]).wait()
            pltpu.make_async_copy(w_hbm.at[pl.ds(off, chunk)], wv[b], isem[b]).wait()

        def issue_gather(b):
            pltpu.async_copy(xt_hbm.at[srcv[b]], rows[b], gsem[b])

        def wait_gather(b):
            pltpu.make_async_copy(xt_hbm.at[srcv[b]], rows[b], gsem[b]).wait()

        def issue_scatter(b):
            pltpu.async_copy(rows[b], acc_sh.at[dstv[b]], ssem[b], add=True)

        def wait_scatter(b):
            pltpu.make_async_copy(rows[b], acc_sh.at[dstv[b]], ssem[b]).wait()

        def multiply(b):
            def rb(i, _):
                for u in range(4):
                    r = i * 4 + u
                    wvec = wv[b][r, :]
                    for k in range(d // LANES):
                        sl = pl.ds(k * LANES, LANES)
                        rows[b][r, sl] = rows[b][r, sl] * wvec
                return 0
            lax.fori_loop(0, chunk // 4, rb, 0)

        def compute(j, b):
            wait_gather(b)

            @pl.when(j >= n_j)
            def _():
                for k in range(chunk):
                    wv[b][k, :] = jnp.zeros((LANES,), jnp.float32)
            multiply(b)
            issue_scatter(b)

        issue_idx(0, 0)
        issue_idx(1, 1)
        issue_idx(2, 2)
        wait_idx(0, 0)
        issue_gather(0)
        wait_idx(1, 1)
        issue_gather(1)
        compute(0, 0)
        wait_idx(2, 2)
        issue_gather(2)
        compute(1, 1)
        wait_scatter(0)
        issue_idx(3, 0)

        def steady(g, _):
            j0 = 2 + 3 * g
            for t in range(3):
                j = j0 + t
                b, b1, b2 = (2 + t) % 3, t % 3, (t + 1) % 3
                wait_idx(j + 1, b1)
                issue_gather(b1)
                compute(j, b)
                wait_scatter(b2)
                issue_idx(j + 2, b2)
            return 0
        lax.fori_loop(0, (nch - 4) // 3, steady, 0)
        bj = (nch - 2) % 3
        wait_idx(nch - 1, (nch - 1) % 3)
        issue_gather((nch - 1) % 3)
        compute(nch - 2, bj)
        compute(nch - 1, (nch - 1) % 3)
        wait_scatter((nch - 3) % 3)
        wait_scatter((nch - 2) % 3)
        wait_scatter((nch - 1) % 3)
        plsc.subcore_barrier()

        for r0, sz in segs:
            a0 = sid * rows_t + r0
            pltpu.sync_copy(acc_sh.at[pl.ds(a0, sz)], zbuf.at[pl.ds(0, sz)])
            pltpu.sync_copy(zbuf.at[pl.ds(0, sz)], out_hbm.at[cid, pl.ds(a0, sz)])

    return agg(xt, src, dst, w)


def kernel(x, edge_index, edge_weight, W1, b1, W2, b2, W_lin, b_lin):
    n, d_in = x.shape
    d_h = W1.shape[0]
    d_out = W2.shape[0]
    n_cls = W_lin.shape[0]
    d1 = 128
    d2 = 128

    src = edge_index[0]
    dst = edge_index[1]

    w1t = jnp.zeros((d_in, d1), jnp.float32).at[:, :d_h].set(W1.T)
    b1row = jnp.zeros((1, d1), jnp.float32).at[0, :d_h].set(b1)
    w2t = jnp.zeros((d1, d2), jnp.float32).at[:d_h, :d_out].set(W2.T)
    b2row = jnp.zeros((1, d2), jnp.float32).at[0, :d_out].set(b2)
    wlt = jnp.zeros((d2, 128), jnp.float32).at[:d_out, :n_cls].set(W_lin.T)

    blrow = jnp.zeros((1, 128), jnp.float32).at[0, :n_cls].set(b_lin)

    n_acc = 10112
    wb = jnp.broadcast_to(edge_weight[:, None], (edge_weight.shape[0], LANES))
    xt1 = _stage_a(x, w1t, b1row, d1)
    p = _sc_agg(xt1, src, dst, wb, n_acc)
    xt2 = _stage_b(p[0], p[1], w2t, b2row, d2, bn=1264)
    q = _sc_agg(xt2, src, dst, wb, n_acc)
    return _stage_c(q[0], q[1], wlt, blrow, n_cls, bn=1264)[:n]


# speedup vs baseline: 6.1834x; 1.0184x over previous
import functools

import jax
import jax.numpy as jnp
from jax import lax
from jax.experimental import pallas as pl
from jax.experimental.pallas import tpu as pltpu
from jax.experimental.pallas import tpu_sc as plsc

MIN_NORM = 1e-15
EPS = 1e-7
MAX_NORM = 1e6

NC = 2
NS = 16
LANES = 16


def _col0_mask(v):
    col = lax.broadcasted_iota(jnp.int32, v.shape, 1)
    return col == 0


def _mask0(v):
    return jnp.where(_col0_mask(v), 0.0, v)


def _sinh(t):
    return 0.5 * (jnp.exp(t) - jnp.exp(-t))


def _arccosh(z):
    inv = 1.0 / z
    return jnp.log(z) + jnp.log1p(jnp.sqrt(jnp.maximum(1.0 - inv * inv, 0.0)))


def _rownorm(y):
    return jnp.maximum(jnp.sqrt(jnp.sum(y * y, axis=1, keepdims=True)), MIN_NORM)


def _proj(x):
    y = _mask0(x)
    x0 = jnp.sqrt(jnp.maximum(1.0 + jnp.sum(y * y, axis=1, keepdims=True), MIN_NORM))
    return jnp.where(_col0_mask(x), x0, x)


def _expmap0_proj(u):
    y = _mask0(u)
    yn = _rownorm(y)
    rest = (_sinh(yn) / yn) * y
    return _proj(rest)


def _logmap0(x):
    y = _mask0(x)
    yn = _rownorm(y)
    theta = jnp.maximum(x[:, 0:1], 1.0 + EPS)
    return (_arccosh(theta) / yn) * y


def _mobius_add_bias(x, hyp_bias):
    u = _logmap0(hyp_bias)
    x0 = x[:, 0:1]
    y = _mask0(x)
    yn = _rownorm(y)
    yu = y / yn
    v = jnp.where(_col0_mask(x), -yn, (1.0 - x0) * yu)
    alpha = jnp.sum(yu * _mask0(u), axis=1, keepdims=True)
    res = u - alpha * v
    ux = jnp.sum(y * _mask0(res), axis=1, keepdims=True)
    pt = jnp.where(_col0_mask(x), ux / jnp.maximum(x0, EPS), res)
    mink = jnp.sum(pt * pt, axis=1, keepdims=True) - 2.0 * pt[:, 0:1] * pt[:, 0:1]
    normu = jnp.minimum(jnp.sqrt(jnp.maximum(mink, EPS)), MAX_NORM)
    theta = jnp.maximum(normu, MIN_NORM)
    ch = 0.5 * (jnp.exp(theta) + jnp.exp(-theta))
    r = ch * x + (_sinh(theta) / theta) * pt
    return _proj(r)


def _hyp_bias(b_row):
    return _expmap0_proj(_mask0(b_row))


def _stage_a_body(x_ref, w1t_ref, b1_ref, o_ref):
    xb = x_ref[...]
    y = _mask0(xb)
    yn = _rownorm(y)
    rest = (_sinh(yn) / yn) * y
    x0 = jnp.sqrt(jnp.maximum(1.0 + jnp.sum(rest * rest, axis=1, keepdims=True),
                              MIN_NORM))
    yn2 = _rownorm(rest)
    theta = jnp.maximum(x0, 1.0 + EPS)
    u = (_arccosh(theta) / yn2) * rest
    mu = jnp.dot(u, w1t_ref[...], preferred_element_type=jnp.float32)
    res = _expmap0_proj(mu)
    res = _mobius_add_bias(res, _hyp_bias(b1_ref[...]))
    o_ref[...] = _logmap0(res)


def _stage_a(x, w1t, b1row, d1, bn=400):
    n = x.shape[0]
    return pl.pallas_call(
        _stage_a_body,
        grid=(n // bn,),
        in_specs=[
            pl.BlockSpec((bn, x.shape[1]), lambda i: (i, 0)),
            pl.BlockSpec(w1t.shape, lambda i: (0, 0)),
            pl.BlockSpec(b1row.shape, lambda i: (0, 0)),
        ],
        out_specs=pl.BlockSpec((bn, d1), lambda i: (i, 0)),
        out_shape=jax.ShapeDtypeStruct((n, d1), jnp.float32),
    )(x, w1t, b1row)


def _stage_b_body(p0_ref, p1_ref, w2t_ref, b2_ref, o_ref):
    support = p0_ref[...] + p1_ref[...]
    h = _expmap0_proj(support)
    xt = jax.nn.relu(_logmap0(h))
    h2 = _expmap0_proj(xt)
    u = _logmap0(h2)
    mu = jnp.dot(u, w2t_ref[...], preferred_element_type=jnp.float32)
    res = _expmap0_proj(mu)
    res = _mobius_add_bias(res, _hyp_bias(b2_ref[...]))
    o_ref[...] = _logmap0(res)


def _stage_b(p0, p1, w2t, b2row, d2, bn=1000):
    n = p0.shape[0]
    return pl.pallas_call(
        _stage_b_body,
        grid=(n // bn,),
        in_specs=[
            pl.BlockSpec((bn, p0.shape[1]), lambda i: (i, 0)),
            pl.BlockSpec((bn, p0.shape[1]), lambda i: (i, 0)),
            pl.BlockSpec(w2t.shape, lambda i: (0, 0)),
            pl.BlockSpec(b2row.shape, lambda i: (0, 0)),
        ],
        out_specs=pl.BlockSpec((bn, d2), lambda i: (i, 0)),
        out_shape=jax.ShapeDtypeStruct((n, d2), jnp.float32),
    )(p0, p1, w2t, b2row)


def _stage_c_body(q0_ref, q1_ref, wlt_ref, bl_ref, o_ref, *, n_cls):
    support = q0_ref[...] + q1_ref[...]
    h = _expmap0_proj(support)
    xt = jax.nn.relu(_logmap0(h))
    h2 = _expmap0_proj(xt)
    ht = _logmap0(h2)
    logits = jnp.dot(ht, wlt_ref[...], preferred_element_type=jnp.float32)
    logits = jax.nn.relu(logits + bl_ref[...])
    col = lax.broadcasted_iota(jnp.int32, logits.shape, 1)
    masked = jnp.where(col < n_cls, logits, -jnp.inf)
    m = jnp.max(masked, axis=1, keepdims=True)
    z = masked - m
    lse = jnp.log(jnp.sum(jnp.where(col < n_cls, jnp.exp(z), 0.0),
                          axis=1, keepdims=True))
    o_ref[...] = (z - lse)[:, :n_cls]


def _stage_c(q0, q1, wlt, blrow, n_cls, bn=1000):
    n = q0.shape[0]
    return pl.pallas_call(
        functools.partial(_stage_c_body, n_cls=n_cls),
        grid=(n // bn,),
        in_specs=[
            pl.BlockSpec((bn, q0.shape[1]), lambda i: (i, 0)),
            pl.BlockSpec((bn, q0.shape[1]), lambda i: (i, 0)),
            pl.BlockSpec(wlt.shape, lambda i: (0, 0)),
            pl.BlockSpec(blrow.shape, lambda i: (0, 0)),
        ],
        out_specs=pl.BlockSpec((bn, n_cls), lambda i: (i, 0)),
        out_shape=jax.ShapeDtypeStruct((n, n_cls), jnp.float32),
    )(q0, q1, wlt, blrow)


def _sc_agg(xt, src, dst, w, n_acc, d_real):
    n, d = xt.shape
    nsl = -(-d_real // LANES)
    e = src.shape[0]
    nw = NC * NS
    chunk = 64
    n_glob = e // chunk
    assert n_glob * chunk == e and n_acc % NS == 0
    base_cnt = n_glob // nw
    extra = n_glob - base_cnt * nw
    nch = base_cnt + (1 if extra else 0)
    nch += (1 - nch) % 3
    assert nch >= 4 and (nch - 4) % 3 == 0
    rows_t = n_acc // NS
    zrows = chunk
    segs = [(r0, min(zrows, rows_t - r0)) for r0 in range(0, rows_t, zrows)]
    assert all(sz % 8 == 0 for _, sz in segs)
    mesh = plsc.VectorSubcoreMesh(core_axis_name="c", subcore_axis_name="s",
                                  num_cores=NC, num_subcores=NS)

    @functools.partial(
        pl.kernel,
        out_type=jax.ShapeDtypeStruct((NC, n_acc, d), jnp.float32),
        mesh=mesh,
        scratch_types=[
            pltpu.VMEM_SHARED((n_acc, d), jnp.float32),
            [pltpu.VMEM((chunk,), jnp.int32)] * 3,
            [pltpu.VMEM((chunk,), jnp.int32)] * 3,
            [pltpu.VMEM((chunk, LANES), jnp.float32)] * 3,
            [pltpu.VMEM((chunk, d), jnp.float32)] * 3,
            [pltpu.SemaphoreType.DMA] * 3,
            [pltpu.SemaphoreType.DMA] * 3,
            [pltpu.SemaphoreType.DMA] * 3,
        ],
    )
    def agg(xt_hbm, src_hbm, dst_hbm, w_hbm, out_hbm,
            acc_sh, srcv, dstv, wv, rows, gsem, isem, ssem):
        zbuf = rows[0]
        cid = lax.axis_index("c")
        sid = lax.axis_index("s")
        wid = sid * NC + cid
        n_j = base_cnt + jnp.where(wid < extra, 1, 0)

        def zrow(i, _):
            for k in range(d // LANES):
                zbuf[i, pl.ds(k * LANES, LANES)] = jnp.zeros((LANES,), jnp.float32)
            return 0
        lax.fori_loop(0, zrows, zrow, 0)
        for r0, sz in segs:
            pltpu.sync_copy(zbuf.at[pl.ds(0, sz)],
                            acc_sh.at[pl.ds(sid * rows_t + r0, sz)])
        plsc.subcore_barrier()

        def off_of(j):
            jj = jnp.where(j < n_j, j, 0)
            return (wid + nw * jj) * chunk

        def issue_idx(j, b):
            off = off_of(j)
            pltpu.async_copy(src_hbm.at[pl.ds(off, chunk)], srcv[b], isem[b])
            pltpu.async_copy(dst_hbm.at[pl.ds(off, chunk)], dstv[b], isem[b])
            pltpu.async_copy(w_hbm.at[pl.ds(off, chunk)], wv[b], isem[b])

        def wait_idx(j, b):
            off = off_of(j)
            pltpu.make_async_copy(src_hbm.at[pl.ds(off, chunk)], srcv[b], isem[b]).wait()
            pltpu.make_async_copy(dst_hbm.at[pl.ds(off, chunk)], dstv[b], isem[b]).wait()
            pltpu.make_async_copy(w_hbm.at[pl.ds(off, chunk)], wv[b], isem[b]).wait()

        def issue_gather(b):
            pltpu.async_copy(xt_hbm.at[srcv[b]], rows[b], gsem[b])

        def wait_gather(b):
            pltpu.make_async_copy(xt_hbm.at[srcv[b]], rows[b], gsem[b]).wait()

        def issue_scatter(b):
            pltpu.async_copy(rows[b], acc_sh.at[dstv[b]], ssem[b], add=True)

        def wait_scatter(b):
            pltpu.make_async_copy(rows[b], acc_sh.at[dstv[b]], ssem[b]).wait()

        def multiply(b):
            def rb(i, _):
                for u in range(4):
                    r = i * 4 + u
                    wvec = wv[b][r, :]
                    for k in range(nsl):
                        sl = pl.ds(k * LANES, LANES)
                        rows[b][r, sl] = rows[b][r, sl] * wvec
                return 0
            lax.fori_loop(0, chunk // 4, rb, 0)

        def compute(j, b):
            wait_gather(b)

            @pl.when(j >= n_j)
            def _():
                for k in range(chunk):
                    wv[b][k, :] = jnp.zeros((LANES,), jnp.float32)
            multiply(b)
            issue_scatter(b)

        issue_idx(0, 0)
        issue_idx(1, 1)
        issue_idx(2, 2)
        wait_idx(0, 0)
        issue_gather(0)
        wait_idx(1, 1)
        issue_gather(1)
        compute(0, 0)
        wait_idx(2, 2)
        issue_gather(2)
        compute(1, 1)
        wait_scatter(0)
        issue_idx(3, 0)

        def steady(g, _):
            j0 = 2 + 3 * g
            for t in range(3):
                j = j0 + t
                b, b1, b2 = (2 + t) % 3, t % 3, (t + 1) % 3
                wait_idx(j + 1, b1)
                issue_gather(b1)
                compute(j, b)
                wait_scatter(b2)
                issue_idx(j + 2, b2)
            return 0
        lax.fori_loop(0, (nch - 4) // 3, steady, 0)
        bj = (nch - 2) % 3
        wait_idx(nch - 1, (nch - 1) % 3)
        issue_gather((nch - 1) % 3)
        compute(nch - 2, bj)
        compute(nch - 1, (nch - 1) % 3)
        wait_scatter((nch - 3) % 3)
        wait_scatter((nch - 2) % 3)
        wait_scatter((nch - 1) % 3)
        plsc.subcore_barrier()

        for r0, sz in segs:
            a0 = sid * rows_t + r0
            pltpu.sync_copy(acc_sh.at[pl.ds(a0, sz)], zbuf.at[pl.ds(0, sz)])
            pltpu.sync_copy(zbuf.at[pl.ds(0, sz)], out_hbm.at[cid, pl.ds(a0, sz)])

    return agg(xt, src, dst, w)


def kernel(x, edge_index, edge_weight, W1, b1, W2, b2, W_lin, b_lin):
    n, d_in = x.shape
    d_h = W1.shape[0]
    d_out = W2.shape[0]
    n_cls = W_lin.shape[0]
    d1 = 128
    d2 = 128

    src = edge_index[0]
    dst = edge_index[1]

    w1t = jnp.zeros((d_in, d1), jnp.float32).at[:, :d_h].set(W1.T)
    b1row = jnp.zeros((1, d1), jnp.float32).at[0, :d_h].set(b1)
    w2t = jnp.zeros((d1, d2), jnp.float32).at[:d_h, :d_out].set(W2.T)
    b2row = jnp.zeros((1, d2), jnp.float32).at[0, :d_out].set(b2)
    wlt = jnp.zeros((d2, 128), jnp.float32).at[:d_out, :n_cls].set(W_lin.T)

    blrow = jnp.zeros((1, 128), jnp.float32).at[0, :n_cls].set(b_lin)

    n_acc = 10112
    wb = jnp.broadcast_to(edge_weight[:, None], (edge_weight.shape[0], LANES))
    xt1 = _stage_a(x, w1t, b1row, d1)
    p = _sc_agg(xt1, src, dst, wb, n_acc, d_h)
    xt2 = _stage_b(p[0], p[1], w2t, b2row, d2, bn=1264)
    q = _sc_agg(xt2, src, dst, wb, n_acc, d_out)
    return _stage_c(q[0], q[1], wlt, blrow, n_cls, bn=1264)[:n]


# speedup vs baseline: 10.7279x; 1.7350x over previous
import functools

import jax
import jax.numpy as jnp
from jax import lax
from jax.experimental import pallas as pl
from jax.experimental.pallas import tpu as pltpu
from jax.experimental.pallas import tpu_sc as plsc

MIN_NORM = 1e-15
EPS = 1e-7
MAX_NORM = 1e6

NC = 2
NS = 16
LANES = 16


def _col0_mask(v):
    col = lax.broadcasted_iota(jnp.int32, v.shape, 1)
    return col == 0


def _mask0(v):
    return jnp.where(_col0_mask(v), 0.0, v)


def _sinh(t):
    return 0.5 * (jnp.exp(t) - jnp.exp(-t))


def _arccosh(z):
    inv = 1.0 / z
    return jnp.log(z) + jnp.log1p(jnp.sqrt(jnp.maximum(1.0 - inv * inv, 0.0)))


def _rownorm(y):
    return jnp.maximum(jnp.sqrt(jnp.sum(y * y, axis=1, keepdims=True)), MIN_NORM)


def _proj(x):
    y = _mask0(x)
    x0 = jnp.sqrt(jnp.maximum(1.0 + jnp.sum(y * y, axis=1, keepdims=True), MIN_NORM))
    return jnp.where(_col0_mask(x), x0, x)


def _expmap0_proj(u):
    y = _mask0(u)
    yn = _rownorm(y)
    rest = (_sinh(yn) / yn) * y
    return _proj(rest)


def _logmap0(x):
    y = _mask0(x)
    yn = _rownorm(y)
    theta = jnp.maximum(x[:, 0:1], 1.0 + EPS)
    return (_arccosh(theta) / yn) * y


def _mobius_add_bias(x, hyp_bias):
    u = _logmap0(hyp_bias)
    x0 = x[:, 0:1]
    y = _mask0(x)
    yn = _rownorm(y)
    yu = y / yn
    v = jnp.where(_col0_mask(x), -yn, (1.0 - x0) * yu)
    alpha = jnp.sum(yu * _mask0(u), axis=1, keepdims=True)
    res = u - alpha * v
    ux = jnp.sum(y * _mask0(res), axis=1, keepdims=True)
    pt = jnp.where(_col0_mask(x), ux / jnp.maximum(x0, EPS), res)
    mink = jnp.sum(pt * pt, axis=1, keepdims=True) - 2.0 * pt[:, 0:1] * pt[:, 0:1]
    normu = jnp.minimum(jnp.sqrt(jnp.maximum(mink, EPS)), MAX_NORM)
    theta = jnp.maximum(normu, MIN_NORM)
    ch = 0.5 * (jnp.exp(theta) + jnp.exp(-theta))
    r = ch * x + (_sinh(theta) / theta) * pt
    return _proj(r)


def _hyp_bias(b_row):
    return _expmap0_proj(_mask0(b_row))


def _stage_a_body(x_ref, w1t_ref, b1_ref, o_ref):
    xb = x_ref[...]
    y = _mask0(xb)
    yn = _rownorm(y)
    rest = (_sinh(yn) / yn) * y
    x0 = jnp.sqrt(jnp.maximum(1.0 + jnp.sum(rest * rest, axis=1, keepdims=True),
                              MIN_NORM))
    yn2 = _rownorm(rest)
    theta = jnp.maximum(x0, 1.0 + EPS)
    u = (_arccosh(theta) / yn2) * rest
    mu = jnp.dot(u, w1t_ref[...], preferred_element_type=jnp.float32)
    res = _expmap0_proj(mu)
    res = _mobius_add_bias(res, _hyp_bias(b1_ref[...]))
    o_ref[...] = _logmap0(res)


def _stage_a(x, w1t, b1row, d1, bn=400):
    n = x.shape[0]
    return pl.pallas_call(
        _stage_a_body,
        grid=(n // bn,),
        in_specs=[
            pl.BlockSpec((bn, x.shape[1]), lambda i: (i, 0)),
            pl.BlockSpec(w1t.shape, lambda i: (0, 0)),
            pl.BlockSpec(b1row.shape, lambda i: (0, 0)),
        ],
        out_specs=pl.BlockSpec((bn, d1), lambda i: (i, 0)),
        out_shape=jax.ShapeDtypeStruct((n, d1), jnp.float32),
    )(x, w1t, b1row)


def _stage_b_body(p0_ref, p1_ref, w2t_ref, b2_ref, o_ref):
    support = p0_ref[...] + p1_ref[...]
    h = _expmap0_proj(support)
    xt = jax.nn.relu(_logmap0(h))
    h2 = _expmap0_proj(xt)
    u = _logmap0(h2)
    mu = jnp.dot(u, w2t_ref[...], preferred_element_type=jnp.float32)
    res = _expmap0_proj(mu)
    res = _mobius_add_bias(res, _hyp_bias(b2_ref[...]))
    o_ref[...] = _logmap0(res)


def _stage_b(p0, p1, w2t, b2row, d2, bn=1000):
    n = p0.shape[0]
    return pl.pallas_call(
        _stage_b_body,
        grid=(n // bn,),
        in_specs=[
            pl.BlockSpec((bn, p0.shape[1]), lambda i: (i, 0)),
            pl.BlockSpec((bn, p0.shape[1]), lambda i: (i, 0)),
            pl.BlockSpec(w2t.shape, lambda i: (0, 0)),
            pl.BlockSpec(b2row.shape, lambda i: (0, 0)),
        ],
        out_specs=pl.BlockSpec((bn, d2), lambda i: (i, 0)),
        out_shape=jax.ShapeDtypeStruct((n, d2), jnp.float32),
    )(p0, p1, w2t, b2row)


def _stage_c_body(q0_ref, q1_ref, wlt_ref, bl_ref, o_ref, *, n_cls):
    support = q0_ref[...] + q1_ref[...]
    h = _expmap0_proj(support)
    xt = jax.nn.relu(_logmap0(h))
    h2 = _expmap0_proj(xt)
    ht = _logmap0(h2)
    logits = jnp.dot(ht, wlt_ref[...], preferred_element_type=jnp.float32)
    logits = jax.nn.relu(logits + bl_ref[...])
    col = lax.broadcasted_iota(jnp.int32, logits.shape, 1)
    masked = jnp.where(col < n_cls, logits, -jnp.inf)
    m = jnp.max(masked, axis=1, keepdims=True)
    z = masked - m
    lse = jnp.log(jnp.sum(jnp.where(col < n_cls, jnp.exp(z), 0.0),
                          axis=1, keepdims=True))
    o_ref[...] = (z - lse)[:, :n_cls]


def _stage_c(q0, q1, wlt, blrow, n_cls, bn=1000):
    n = q0.shape[0]
    return pl.pallas_call(
        functools.partial(_stage_c_body, n_cls=n_cls),
        grid=(n // bn,),
        in_specs=[
            pl.BlockSpec((bn, q0.shape[1]), lambda i: (i, 0)),
            pl.BlockSpec((bn, q0.shape[1]), lambda i: (i, 0)),
            pl.BlockSpec(wlt.shape, lambda i: (0, 0)),
            pl.BlockSpec(blrow.shape, lambda i: (0, 0)),
        ],
        out_specs=pl.BlockSpec((bn, n_cls), lambda i: (i, 0)),
        out_shape=jax.ShapeDtypeStruct((n, n_cls), jnp.float32),
    )(q0, q1, wlt, blrow)


def _sc_agg(xt, src, dst, w, n_acc, d_real):
    n, d = xt.shape
    nsl = -(-d_real // LANES)
    e = src.shape[0]
    nw = NC * NS
    chunk = 80
    n_glob = e // chunk
    assert n_glob * chunk == e and n_acc % NS == 0
    base_cnt = n_glob // nw
    extra = n_glob - base_cnt * nw
    nch = base_cnt + (1 if extra else 0)
    nch += (1 - nch) % 3
    assert nch >= 4 and (nch - 4) % 3 == 0
    rows_t = n_acc // NS
    zrows = chunk
    segs = [(r0, min(zrows, rows_t - r0)) for r0 in range(0, rows_t, zrows)]
    assert all(sz % 8 == 0 for _, sz in segs)
    mesh = plsc.VectorSubcoreMesh(core_axis_name="c", subcore_axis_name="s",
                                  num_cores=NC, num_subcores=NS)

    @functools.partial(
        pl.kernel,
        out_type=jax.ShapeDtypeStruct((NC, n_acc, d), jnp.float32),
        mesh=mesh,
        scratch_types=[
            pltpu.VMEM_SHARED((n_acc, d), jnp.float32),
            [pltpu.VMEM((chunk,), jnp.int32)] * 3,
            [pltpu.VMEM((chunk,), jnp.int32)] * 3,
            [pltpu.VMEM((chunk,), jnp.float32)] * 3,
            [pltpu.VMEM((chunk, d), jnp.float32)] * 3,
            [pltpu.SemaphoreType.DMA] * 3,
            [pltpu.SemaphoreType.DMA] * 3,
            [pltpu.SemaphoreType.DMA] * 3,
        ],
    )
    def agg(xt_hbm, src_hbm, dst_hbm, w_hbm, out_hbm,
            acc_sh, srcv, dstv, wv, rows, gsem, isem, ssem):
        zbuf = rows[0]
        cid = lax.axis_index("c")
        sid = lax.axis_index("s")
        wid = sid * NC + cid
        n_j = base_cnt + jnp.where(wid < extra, 1, 0)

        def zrow(i, _):
            for k in range(d // LANES):
                zbuf[i, pl.ds(k * LANES, LANES)] = jnp.zeros((LANES,), jnp.float32)
            return 0
        lax.fori_loop(0, zrows, zrow, 0)
        for r0, sz in segs:
            pltpu.sync_copy(zbuf.at[pl.ds(0, sz)],
                            acc_sh.at[pl.ds(sid * rows_t + r0, sz)])
        plsc.subcore_barrier()

        def off_of(j):
            jj = jnp.where(j < n_j, j, 0)
            return (wid + nw * jj) * chunk

        def issue_idx(j, b):
            off = off_of(j)
            pltpu.async_copy(src_hbm.at[pl.ds(off, chunk)], srcv[b], isem[b])
            pltpu.async_copy(dst_hbm.at[pl.ds(off, chunk)], dstv[b], isem[b])
            pltpu.async_copy(w_hbm.at[pl.ds(off, chunk)], wv[b], isem[b])

        def wait_idx(j, b):
            off = off_of(j)
            pltpu.make_async_copy(src_hbm.at[pl.ds(off, chunk)], srcv[b], isem[b]).wait()
            pltpu.make_async_copy(dst_hbm.at[pl.ds(off, chunk)], dstv[b], isem[b]).wait()
            pltpu.make_async_copy(w_hbm.at[pl.ds(off, chunk)], wv[b], isem[b]).wait()

        def issue_gather(b):
            pltpu.async_copy(xt_hbm.at[srcv[b]], rows[b], gsem[b])

        def wait_gather(b):
            pltpu.make_async_copy(xt_hbm.at[srcv[b]], rows[b], gsem[b]).wait()

        def issue_scatter(b):
            pltpu.async_copy(rows[b], acc_sh.at[dstv[b]], ssem[b], add=True)

        def wait_scatter(b):
            pltpu.make_async_copy(rows[b], acc_sh.at[dstv[b]], ssem[b]).wait()

        def multiply(b):
            def rb(i, _):
                wgrp = wv[b][pl.ds(i * LANES, LANES)]
                for u in range(LANES):
                    r = i * LANES + u
                    wval = wgrp[u]
                    for k in range(nsl):
                        sl = pl.ds(k * LANES, LANES)
                        rows[b][r, sl] = rows[b][r, sl] * wval
                return 0
            lax.fori_loop(0, chunk // LANES, rb, 0)

        def compute(j, b):
            wait_gather(b)

            @pl.when(j >= n_j)
            def _():
                for k in range(chunk // LANES):
                    wv[b][pl.ds(k * LANES, LANES)] = jnp.zeros((LANES,), jnp.float32)
            multiply(b)
            issue_scatter(b)

        issue_idx(0, 0)
        issue_idx(1, 1)
        issue_idx(2, 2)
        wait_idx(0, 0)
        issue_gather(0)
        wait_idx(1, 1)
        issue_gather(1)
        compute(0, 0)
        wait_idx(2, 2)
        issue_gather(2)
        compute(1, 1)
        wait_scatter(0)
        issue_idx(3, 0)

        def steady(g, _):
            j0 = 2 + 3 * g
            for t in range(3):
                j = j0 + t
                b, b1, b2 = (2 + t) % 3, t % 3, (t + 1) % 3
                wait_idx(j + 1, b1)
                issue_gather(b1)
                compute(j, b)
                wait_scatter(b2)
                issue_idx(j + 2, b2)
            return 0
        lax.fori_loop(0, (nch - 4) // 3, steady, 0)
        bj = (nch - 2) % 3
        wait_idx(nch - 1, (nch - 1) % 3)
        issue_gather((nch - 1) % 3)
        compute(nch - 2, bj)
        compute(nch - 1, (nch - 1) % 3)
        wait_scatter((nch - 3) % 3)
        wait_scatter((nch - 2) % 3)
        wait_scatter((nch - 1) % 3)
        plsc.subcore_barrier()

        for r0, sz in segs:
            a0 = sid * rows_t + r0
            pltpu.sync_copy(acc_sh.at[pl.ds(a0, sz)], zbuf.at[pl.ds(0, sz)])
            pltpu.sync_copy(zbuf.at[pl.ds(0, sz)], out_hbm.at[cid, pl.ds(a0, sz)])

    return agg(xt, src, dst, w)


def kernel(x, edge_index, edge_weight, W1, b1, W2, b2, W_lin, b_lin):
    n, d_in = x.shape
    d_h = W1.shape[0]
    d_out = W2.shape[0]
    n_cls = W_lin.shape[0]
    d1 = 128
    d2 = 128

    src = edge_index[0]
    dst = edge_index[1]

    w1t = jnp.zeros((d_in, d1), jnp.float32).at[:, :d_h].set(W1.T)
    b1row = jnp.zeros((1, d1), jnp.float32).at[0, :d_h].set(b1)
    w2t = jnp.zeros((d1, d2), jnp.float32).at[:d_h, :d_out].set(W2.T)
    b2row = jnp.zeros((1, d2), jnp.float32).at[0, :d_out].set(b2)
    wlt = jnp.zeros((d2, 128), jnp.float32).at[:d_out, :n_cls].set(W_lin.T)

    blrow = jnp.zeros((1, 128), jnp.float32).at[0, :n_cls].set(b_lin)

    n_acc = 10112
    xt1 = _stage_a(x, w1t, b1row, d1)
    p = _sc_agg(xt1, src, dst, edge_weight, n_acc, d_h)
    xt2 = _stage_b(p[0], p[1], w2t, b2row, d2, bn=1264)
    q = _sc_agg(xt2, src, dst, edge_weight, n_acc, d_out)
    return _stage_c(q[0], q[1], wlt, blrow, n_cls, bn=1264)[:n]
